# baseline (device time: 22813 ns/iter reference)
import jax
import jax.numpy as jnp
from jax import lax
from jax.experimental import pallas as pl
from jax.experimental.pallas import tpu as pltpu

N_DEV = 32
N_TOK = 2048
N_EXP = 128
CAP = 12
E_LOC = N_EXP // N_DEV
SLOTS = E_LOC * CAP
TOK_LOC = N_TOK // N_DEV
D_IN = 512
D_OUT = 1024


def _lane_cumsum(x):
    c = x
    lane = lax.broadcasted_iota(jnp.int32, x.shape, 1)
    k = 1
    while k < x.shape[1]:
        c = c + jnp.where(lane >= k, jnp.roll(c, k, axis=1), 0)
        k *= 2
    return c


def _body(x_ref, w_ref, e_row_ref, out_ref,
          xb_ref, xg_ref, y_ref, meta_vmem, meta_smem, send_sems, recv_sem,
          meta_sem, gather_sems):
    me = lax.axis_index("i")

    out_ref[...] = jnp.zeros_like(out_ref)

    bar = pltpu.get_barrier_semaphore()
    for off in range(1, N_DEV):
        pl.semaphore_signal(
            bar, inc=1,
            device_id=((me + off) % N_DEV,),
            device_id_type=pl.DeviceIdType.MESH,
        )

    e_row = e_row_ref[:, :]
    le_col = lax.broadcasted_iota(jnp.int32, (E_LOC, 1), 0)
    mask4 = (e_row == me * E_LOC + le_col).astype(jnp.int32)
    cum4 = _lane_cumsum(mask4)

    s_iota = lax.broadcasted_iota(jnp.int32, (SLOTS, 1), 0)
    r = s_iota % CAP
    cum = jnp.reshape(
        jnp.broadcast_to(cum4[:, None, :], (E_LOC, CAP, N_TOK)),
        (SLOTS, N_TOK))
    total = cum[:, N_TOK - 1:N_TOK]
    valid = r < total

    lane = lax.broadcasted_iota(jnp.int32, (SLOTS, N_TOK), 1)
    em = jnp.reshape(
        jnp.broadcast_to(mask4[:, None, :], (E_LOC, CAP, N_TOK)),
        (SLOTS, N_TOK))
    hit = (em == 1) & (cum == r + 1)
    tok = jnp.min(jnp.where(hit, lane, 2 * N_TOK), axis=1, keepdims=True)
    tok_enc = jnp.where(valid, tok, -1)

    e_col = lax.broadcasted_iota(jnp.int32, (N_EXP, 1), 0)
    ee = (e_row == e_col)
    lane2 = lax.broadcasted_iota(jnp.int32, (N_EXP, N_TOK), 1)
    start = me * TOK_LOC
    w_enc = jnp.where(lane2 < start, 4096, 0) + \
        jnp.where(lane2 < start + TOK_LOC, 1, 0)
    enc = jnp.sum(jnp.where(ee, w_enc, 0), axis=1, keepdims=True)
    a = enc >> 12
    b = enc & 4095
    kept = jnp.maximum(jnp.minimum(b, CAP) - jnp.minimum(a, CAP), 0)
    nrecv = jnp.sum(kept, axis=0, keepdims=True)

    meta_vmem[0:SLOTS, :] = tok_enc
    meta_vmem[SLOTS:SLOTS + 1, :] = nrecv
    cp = pltpu.make_async_copy(meta_vmem, meta_smem, meta_sem)
    cp.start()
    cp.wait()

    gathers = []
    for s in range(SLOTS):
        t = jnp.maximum(meta_smem[s, 0], 0)
        g = pltpu.make_async_copy(
            x_ref.at[pl.ds((t // 8) * 8, 8), :],
            xb_ref.at[s],
            gather_sems.at[s])
        g.start()
        gathers.append((t, g))
    for s, (t, g) in enumerate(gathers):
        g.wait()
        xg_ref[pl.ds(s, 1), :] = xb_ref[s, pl.ds(lax.rem(t, 8), 1), :]

    for l in range(E_LOC):
        y_ref[l * CAP:(l + 1) * CAP, :] = jnp.dot(
            xg_ref[l * CAP:(l + 1) * CAP, :], w_ref[l],
            preferred_element_type=jnp.float32,
        )

    pl.semaphore_wait(bar, N_DEV - 1)

    descs = []
    for s in range(SLOTS):
        t = meta_smem[s, 0]
        descs.append((t, pltpu.make_async_remote_copy(
            src_ref=y_ref.at[pl.ds(s, 1), :],
            dst_ref=out_ref.at[pl.ds(lax.rem(t, TOK_LOC), 1), :],
            send_sem=send_sems.at[s],
            recv_sem=recv_sem,
            device_id=(lax.div(t, TOK_LOC),),
            device_id_type=pl.DeviceIdType.MESH,
        )))
    for s in range(SLOTS):
        @pl.when(descs[s][0] >= 0)
        def _(s=s):
            descs[s][1].start()
    for s in range(SLOTS):
        @pl.when(descs[s][0] >= 0)
        def _(s=s):
            descs[s][1].wait_send()

    recv_d = pltpu.make_async_remote_copy(
        src_ref=y_ref.at[pl.ds(0, 1), :],
        dst_ref=out_ref.at[pl.ds(0, 1), :],
        send_sem=send_sems.at[0],
        recv_sem=recv_sem,
        device_id=(me,),
        device_id_type=pl.DeviceIdType.MESH,
    )

    def rbody(i, carry):
        recv_d.wait_recv()
        return carry

    lax.fori_loop(0, meta_smem[SLOTS, 0], rbody, 0)


def kernel(x, router_W, route_idx, expert_W):
    del router_W

    e_row = route_idx.reshape(1, N_TOK)

    return pl.pallas_call(
        _body,
        out_shape=jax.ShapeDtypeStruct((TOK_LOC, D_OUT), jnp.float32),
        in_specs=[
            pl.BlockSpec(memory_space=pltpu.MemorySpace.HBM),
            pl.BlockSpec(memory_space=pltpu.VMEM),
            pl.BlockSpec(memory_space=pltpu.VMEM),
        ],
        out_specs=pl.BlockSpec(memory_space=pltpu.VMEM),
        scratch_shapes=[
            pltpu.VMEM((SLOTS, 8, D_IN), jnp.float32),
            pltpu.VMEM((SLOTS, D_IN), jnp.float32),
            pltpu.VMEM((SLOTS, D_OUT), jnp.float32),
            pltpu.VMEM((SLOTS + 1, 1), jnp.int32),
            pltpu.SMEM((SLOTS + 1, 1), jnp.int32),
            pltpu.SemaphoreType.DMA((SLOTS,)),
            pltpu.SemaphoreType.DMA,
            pltpu.SemaphoreType.DMA,
            pltpu.SemaphoreType.DMA((SLOTS,)),
        ],
        compiler_params=pltpu.CompilerParams(collective_id=0),
    )(x, expert_W, e_row)


# device time: 20322 ns/iter; 1.1226x vs baseline; 1.1226x over previous
import jax
import jax.numpy as jnp
from jax import lax
from jax.experimental import pallas as pl
from jax.experimental.pallas import tpu as pltpu

N_DEV = 32
N_TOK = 2048
N_EXP = 128
CAP = 12
E_LOC = N_EXP // N_DEV
SLOTS = E_LOC * CAP
TOK_LOC = N_TOK // N_DEV
D_IN = 512
D_OUT = 1024


def _lane_cumsum(x):
    c = x
    lane = lax.broadcasted_iota(jnp.int32, x.shape, 1)
    k = 1
    while k < x.shape[1]:
        c = c + jnp.where(lane >= k, jnp.roll(c, k, axis=1), 0)
        k *= 2
    return c


def _body(x_ref, w_ref, e_row_ref, out_ref,
          xg_ref, y_ref, meta_vmem, meta_smem, send_sems, recv_sem,
          meta_sem):
    me = lax.axis_index("i")

    out_ref[...] = jnp.zeros_like(out_ref)

    bar = pltpu.get_barrier_semaphore()
    for off in range(1, N_DEV):
        pl.semaphore_signal(
            bar, inc=1,
            device_id=((me + off) % N_DEV,),
            device_id_type=pl.DeviceIdType.MESH,
        )

    e_row = e_row_ref[:, :]
    le_col = lax.broadcasted_iota(jnp.int32, (E_LOC, 1), 0)
    mask4 = (e_row == me * E_LOC + le_col).astype(jnp.int32)
    cum4 = _lane_cumsum(mask4)

    s_iota = lax.broadcasted_iota(jnp.int32, (SLOTS, 1), 0)
    r = s_iota % CAP
    cum = jnp.reshape(
        jnp.broadcast_to(cum4[:, None, :], (E_LOC, CAP, N_TOK)),
        (SLOTS, N_TOK))
    total = cum[:, N_TOK - 1:N_TOK]
    valid = r < total

    lane = lax.broadcasted_iota(jnp.int32, (SLOTS, N_TOK), 1)
    em = jnp.reshape(
        jnp.broadcast_to(mask4[:, None, :], (E_LOC, CAP, N_TOK)),
        (SLOTS, N_TOK))
    hit = (em == 1) & (cum == r + 1)
    tok = jnp.min(jnp.where(hit, lane, 2 * N_TOK), axis=1, keepdims=True)
    tok_enc = jnp.where(valid, tok, -1)

    e_col = lax.broadcasted_iota(jnp.int32, (N_EXP, 1), 0)
    ee = (e_row == e_col)
    lane2 = lax.broadcasted_iota(jnp.int32, (N_EXP, N_TOK), 1)
    start = me * TOK_LOC
    w_enc = jnp.where(lane2 < start, 4096, 0) + \
        jnp.where(lane2 < start + TOK_LOC, 1, 0)
    enc = jnp.sum(jnp.where(ee, w_enc, 0), axis=1, keepdims=True)
    a = enc >> 12
    b = enc & 4095
    kept = jnp.maximum(jnp.minimum(b, CAP) - jnp.minimum(a, CAP), 0)
    nrecv = jnp.sum(kept, axis=0, keepdims=True)

    meta_vmem[0:SLOTS, :] = tok_enc
    meta_vmem[SLOTS:SLOTS + 1, :] = nrecv
    cp = pltpu.make_async_copy(meta_vmem, meta_smem, meta_sem)
    cp.start()
    cp.wait()

    for s in range(SLOTS):
        t = jnp.maximum(meta_smem[s, 0], 0)
        xg_ref[pl.ds(s, 1), :] = x_ref[pl.ds(t, 1), :]

    for l in range(E_LOC):
        y_ref[l * CAP:(l + 1) * CAP, :] = jnp.dot(
            xg_ref[l * CAP:(l + 1) * CAP, :], w_ref[l],
            preferred_element_type=jnp.float32,
        )

    pl.semaphore_wait(bar, N_DEV - 1)

    descs = []
    for s in range(SLOTS):
        t = meta_smem[s, 0]
        descs.append((t, pltpu.make_async_remote_copy(
            src_ref=y_ref.at[pl.ds(s, 1), :],
            dst_ref=out_ref.at[pl.ds(lax.rem(t, TOK_LOC), 1), :],
            send_sem=send_sems.at[s],
            recv_sem=recv_sem,
            device_id=(lax.div(t, TOK_LOC),),
            device_id_type=pl.DeviceIdType.MESH,
        )))
    for s in range(SLOTS):
        @pl.when(descs[s][0] >= 0)
        def _(s=s):
            descs[s][1].start()
    for s in range(SLOTS):
        @pl.when(descs[s][0] >= 0)
        def _(s=s):
            descs[s][1].wait_send()

    recv_d = pltpu.make_async_remote_copy(
        src_ref=y_ref.at[pl.ds(0, 1), :],
        dst_ref=out_ref.at[pl.ds(0, 1), :],
        send_sem=send_sems.at[0],
        recv_sem=recv_sem,
        device_id=(me,),
        device_id_type=pl.DeviceIdType.MESH,
    )

    def rbody(i, carry):
        recv_d.wait_recv()
        return carry

    lax.fori_loop(0, meta_smem[SLOTS, 0], rbody, 0)


def kernel(x, router_W, route_idx, expert_W):
    del router_W

    e_row = route_idx.reshape(1, N_TOK)

    return pl.pallas_call(
        _body,
        out_shape=jax.ShapeDtypeStruct((TOK_LOC, D_OUT), jnp.float32),
        in_specs=[
            pl.BlockSpec(memory_space=pltpu.VMEM),
            pl.BlockSpec(memory_space=pltpu.VMEM),
            pl.BlockSpec(memory_space=pltpu.VMEM),
        ],
        out_specs=pl.BlockSpec(memory_space=pltpu.VMEM),
        scratch_shapes=[
            pltpu.VMEM((SLOTS, D_IN), jnp.float32),
            pltpu.VMEM((SLOTS, D_OUT), jnp.float32),
            pltpu.VMEM((SLOTS + 1, 1), jnp.int32),
            pltpu.SMEM((SLOTS + 1, 1), jnp.int32),
            pltpu.SemaphoreType.DMA((SLOTS,)),
            pltpu.SemaphoreType.DMA,
            pltpu.SemaphoreType.DMA,
        ],
        compiler_params=pltpu.CompilerParams(collective_id=0),
    )(x, expert_W, e_row)
